# Initial kernel scaffold; baseline (speedup 1.0000x reference)
import jax
import jax.numpy as jnp
from jax.experimental import pallas as pl


def kernel(hidden_states, gate_w, Wg, Wu, Wd, Ag, Bg, Au, Bu, Ad, Bd):
    raise NotImplementedError("write your pallas kernel here")


# speedup vs baseline: 5.9163x; 5.9163x over previous
import functools

import jax
import jax.numpy as jnp
from jax.experimental import pallas as pl

NE = 8
RK = 16
KTOP = 2
SCALE = 2.0
BF = jnp.bfloat16
F32 = jnp.float32


def _moe_body(x_ref, gw_ref, wg_ref, wu_ref, wd_ref, ag_ref, au_ref,
              bg_ref, bu_ref, ad_ref, bd_ref, o_ref):
    xb = x_ref[...]
    x16 = xb.astype(BF)

    logits = jnp.dot(xb, gw_ref[...], preferred_element_type=F32)
    idx = jax.lax.broadcasted_iota(jnp.int32, logits.shape, 1)
    m1 = jnp.max(logits, axis=-1, keepdims=True)
    e0 = jnp.min(jnp.where(logits >= m1, idx, NE), axis=-1, keepdims=True)
    l2 = jnp.where(idx == e0, -1e30, logits)
    m2 = jnp.max(l2, axis=-1, keepdims=True)
    e1 = jnp.min(jnp.where(l2 >= m2, idx, NE), axis=-1, keepdims=True)
    w0 = jax.nn.sigmoid(m1 - m2)
    w1 = 1.0 - w0

    ce = jax.lax.broadcasted_iota(jnp.int32, (xb.shape[0], NE * RK), 1) // RK
    mk0 = (ce == e0).astype(F32)
    mk1 = (ce == e1).astype(F32)

    cg = jnp.dot(x16, wg_ref[...], preferred_element_type=F32)
    cu = jnp.dot(x16, wu_ref[...], preferred_element_type=F32)
    ug = jnp.dot(x16, ag_ref[...], preferred_element_type=F32)
    uu = jnp.dot(x16, au_ref[...], preferred_element_type=F32)

    def slot(mk, w):
        dg = jnp.dot((ug * mk).astype(BF), bg_ref[...], preferred_element_type=F32)
        du = jnp.dot((uu * mk).astype(BF), bu_ref[...], preferred_element_type=F32)
        g = cg + SCALE * dg
        u = cu + SCALE * du
        z = (g * jax.nn.sigmoid(g)) * u * w
        v = jnp.dot(z.astype(BF), ad_ref[...], preferred_element_type=F32) * mk
        return z, v

    z0, v0 = slot(mk0, w0)
    z1, v1 = slot(mk1, w1)
    out = jnp.dot((z0 + z1).astype(BF), wd_ref[...], preferred_element_type=F32)
    out = out + SCALE * jnp.dot((v0 + v1).astype(BF), bd_ref[...],
                                preferred_element_type=F32)
    o_ref[...] = out


@functools.partial(jax.jit, static_argnames=("interpret",))
def _run(x, gwT, WgT, WuT, WdT, AgT, AuT, BgT, BuT, AdT, BdT, interpret=False):
    T, D = x.shape
    FF = WgT.shape[1]
    TT = 256
    const = lambda shape: pl.BlockSpec(shape, lambda i: (0, 0))
    return pl.pallas_call(
        _moe_body,
        grid=(T // TT,),
        in_specs=[
            pl.BlockSpec((TT, D), lambda i: (i, 0)),
            const((D, NE)),
            const((D, FF)), const((D, FF)), const((FF, D)),
            const((D, NE * RK)), const((D, NE * RK)),
            const((NE * RK, FF)), const((NE * RK, FF)),
            const((FF, NE * RK)), const((NE * RK, D)),
        ],
        out_specs=pl.BlockSpec((TT, D), lambda i: (i, 0)),
        out_shape=jax.ShapeDtypeStruct((T, D), F32),
        interpret=interpret,
    )(x, gwT, WgT, WuT, WdT, AgT, AuT, BgT, BuT, AdT, BdT)


def kernel(hidden_states, gate_w, Wg, Wu, Wd, Ag, Bg, Au, Bu, Ad, Bd,
           interpret=False):
    B, S, D = hidden_states.shape
    x = hidden_states.reshape(B * S, D)
    gwT = gate_w.T
    WgT = Wg.T.astype(BF)
    WuT = Wu.T.astype(BF)
    WdT = Wd.T.astype(BF)
    AgT = Ag.transpose(2, 0, 1).reshape(D, NE * RK).astype(BF)
    AuT = Au.transpose(2, 0, 1).reshape(D, NE * RK).astype(BF)
    BgT = Bg.transpose(0, 2, 1).reshape(NE * RK, -1).astype(BF)
    BuT = Bu.transpose(0, 2, 1).reshape(NE * RK, -1).astype(BF)
    AdT = Ad.transpose(2, 0, 1).reshape(-1, NE * RK).astype(BF)
    BdT = Bd.transpose(0, 2, 1).reshape(NE * RK, D).astype(BF)
    out = _run(x, gwT, WgT, WuT, WdT, AgT, AuT, BgT, BuT, AdT, BdT,
               interpret=interpret)
    return out.reshape(B, S, D)


# speedup vs baseline: 6.8233x; 1.1533x over previous
import jax
import jax.numpy as jnp
from jax.experimental import pallas as pl

NE = 8
RK = 16
SCALE = 2.0
BF = jnp.bfloat16
F32 = jnp.float32

_DNT = (((1,), (1,)), ((), ()))


def _moe_body(x_ref, gw_ref, wg_ref, wu_ref, wd_ref, ag_ref, au_ref,
              bg_ref, bu_ref, ad_ref, bd_ref, o_ref):
    x32 = x_ref[...]
    xb = x32.astype(BF)
    TT = xb.shape[0]

    logits = jax.lax.dot_general(x32, gw_ref[...], _DNT,
                                 preferred_element_type=F32)
    idx = jax.lax.broadcasted_iota(jnp.int32, logits.shape, 1)
    m1 = jnp.max(logits, axis=-1, keepdims=True)
    e0 = jnp.min(jnp.where(logits >= m1, idx, NE), axis=-1, keepdims=True)
    l2 = jnp.where(idx == e0, -1e30, logits)
    m2 = jnp.max(l2, axis=-1, keepdims=True)
    e1 = jnp.min(jnp.where(l2 >= m2, idx, NE), axis=-1, keepdims=True)
    w0 = jax.nn.sigmoid(m1 - m2)
    w1 = 1.0 - w0

    ce = jax.lax.broadcasted_iota(jnp.int32, (TT, NE * RK), 1) // RK
    mk0 = (ce == e0).astype(BF)
    mk1 = (ce == e1).astype(BF)

    cg = jax.lax.dot_general(xb, wg_ref[...], _DNT,
                             preferred_element_type=F32)
    cu = jax.lax.dot_general(xb, wu_ref[...], _DNT,
                             preferred_element_type=F32)
    ug = jax.lax.dot_general(xb, ag_ref[...], _DNT,
                             preferred_element_type=F32).astype(BF)
    uu = jax.lax.dot_general(xb, au_ref[...], _DNT,
                             preferred_element_type=F32).astype(BF)

    ugm = jnp.concatenate([ug * mk0, ug * mk1], axis=0)
    uum = jnp.concatenate([uu * mk0, uu * mk1], axis=0)
    dg = jnp.dot(ugm, bg_ref[...], preferred_element_type=F32)
    du = jnp.dot(uum, bu_ref[...], preferred_element_type=F32)

    def slot(k, mk, w):
        g = (cg + SCALE * dg[k * TT:(k + 1) * TT]).astype(BF)
        u = (cu + SCALE * du[k * TT:(k + 1) * TT]).astype(BF)
        z = (g * jax.nn.sigmoid(g)) * u * w.astype(BF)
        v = jax.lax.dot_general(z, ad_ref[...], _DNT,
                                preferred_element_type=F32)
        return z, (v * mk.astype(F32)).astype(BF)

    z0, v0 = slot(0, mk0, w0)
    z1, v1 = slot(1, mk1, w1)
    out = jax.lax.dot_general(z0 + z1, wd_ref[...], _DNT,
                              preferred_element_type=F32)
    out = out + SCALE * jnp.dot(v0 + v1, bd_ref[...],
                                preferred_element_type=F32)
    o_ref[...] = out


@jax.jit
def _run(x, gate_w, Wg16, Wu16, Wd16, Ag2, Au2, BgT, BuT, Ad2, BdT):
    T, D = x.shape
    FF = Wg16.shape[0]
    TT = 256
    const = lambda shape: pl.BlockSpec(shape, lambda i: (0, 0))
    return pl.pallas_call(
        _moe_body,
        grid=(T // TT,),
        in_specs=[
            pl.BlockSpec((TT, D), lambda i: (i, 0)),
            const((NE, D)),
            const((FF, D)), const((FF, D)), const((D, FF)),
            const((NE * RK, D)), const((NE * RK, D)),
            const((NE * RK, FF)), const((NE * RK, FF)),
            const((NE * RK, FF)), const((NE * RK, D)),
        ],
        out_specs=pl.BlockSpec((TT, D), lambda i: (i, 0)),
        out_shape=jax.ShapeDtypeStruct((T, D), F32),
    )(x, gate_w, Wg16, Wu16, Wd16, Ag2, Au2, BgT, BuT, Ad2, BdT)


def kernel(hidden_states, gate_w, Wg, Wu, Wd, Ag, Bg, Au, Bu, Ad, Bd):
    B, S, D = hidden_states.shape
    x = hidden_states.reshape(B * S, D)
    Wg16 = Wg.astype(BF)
    Wu16 = Wu.astype(BF)
    Wd16 = Wd.astype(BF)
    Ag2 = Ag.reshape(NE * RK, -1).astype(BF)
    Au2 = Au.reshape(NE * RK, -1).astype(BF)
    Ad2 = Ad.reshape(NE * RK, -1).astype(BF)
    BgT = Bg.transpose(0, 2, 1).reshape(NE * RK, -1).astype(BF)
    BuT = Bu.transpose(0, 2, 1).reshape(NE * RK, -1).astype(BF)
    BdT = Bd.transpose(0, 2, 1).reshape(NE * RK, -1).astype(BF)
    out = _run(x, gate_w, Wg16, Wu16, Wd16, Ag2, Au2, BgT, BuT, Ad2, BdT)
    return out.reshape(B, S, D)
